# baseline (speedup 1.0000x reference)
import functools

import jax
import jax.numpy as jnp
from jax import lax
from jax.experimental import pallas as pl
from jax.experimental.pallas import tpu as pltpu
from jax.experimental.pallas import tpu_sc as plsc

N_LABELS = 10
N_NEURONS = 6400
BATCH = 4096

TC_ROWS = 2816
SC_ROWS = BATCH - TC_ROWS

NC = 2
NS = 16
NW = NC * NS
ROWS_PER_W = SC_ROWS // NW
CHUNK = 8
NCHUNKS = ROWS_PER_W // CHUNK
NVREG = N_NEURONS // 16

BM = 352
BK = 3200
NKB = N_NEURONS // BK

_mesh = plsc.VectorSubcoreMesh(
    core_axis_name="c", subcore_axis_name="s", num_cores=NC, num_subcores=NS
)


@functools.partial(
    pl.kernel,
    out_type=jax.ShapeDtypeStruct((SC_ROWS,), jnp.int32),
    mesh=_mesh,
    scratch_types=[
        pltpu.VMEM((N_NEURONS,), jnp.int32),
        pltpu.VMEM((2, CHUNK, N_NEURONS), jnp.float32),
        pltpu.VMEM((CHUNK * 256,), jnp.float32),
        pltpu.VMEM((256,), jnp.float32),
        pltpu.VMEM((ROWS_PER_W,), jnp.int32),
        pltpu.SemaphoreType.DMA,
        pltpu.SemaphoreType.DMA,
    ],
    compiler_params=pltpu.CompilerParams(needs_layout_passes=False),
)
def _voting_kernel(
    fr_hbm, asn_hbm, out_hbm, lbl_v, buf_v, acc_v, tmp_v, out_v, sem0, sem1
):
    wid = lax.axis_index("s") * NC + lax.axis_index("c")
    iota = lax.iota(jnp.int32, 16)
    zeros = jnp.zeros((16,), jnp.float32)
    ones = jnp.ones((16,), jnp.float32)
    sems = (sem0, sem1)

    def start_dma(c, b):
        row0 = TC_ROWS + wid * ROWS_PER_W + c * CHUNK
        return pltpu.async_copy(
            fr_hbm.at[pl.ds(row0, CHUNK)], buf_v.at[b], sems[b]
        )

    def wait_dma(b):
        pltpu.make_async_copy(
            fr_hbm.at[pl.ds(0, CHUNK)], buf_v.at[b], sems[b]
        ).wait()

    pltpu.sync_copy(asn_hbm, lbl_v)
    start_dma(0, 0)
    start_dma(1, 1)

    for l in range(16):
        tmp_v[pl.ds(l * 16, 16)] = zeros
    for r in range(CHUNK * 16):
        acc_v[pl.ds(r * 16, 16)] = zeros

    @pl.loop(0, NVREG)
    def _count(j):
        idx16 = lbl_v[pl.ds(j * 16, 16)] * 16 + iota
        plsc.addupdate_scatter(tmp_v, [idx16], ones)
        lbl_v[pl.ds(j * 16, 16)] = idx16

    cnt = zeros
    for l in range(16):
        row = tmp_v[pl.ds(l * 16, 16)]
        plsc.store_scatter(acc_v, [iota * 16 + l], row)
    for l in range(16):
        cnt = cnt + acc_v[pl.ds(l * 16, 16)]
    safe_cnt = jnp.maximum(cnt, 1.0)
    lane_ok = iota < N_LABELS
    cnt_pos = cnt > 0.0

    for l in range(16):
        acc_v[pl.ds(l * 16, 16)] = zeros

    def process(c, b):
        @pl.loop(0, NVREG, unroll=4)
        def _cols(j):
            idx16 = lbl_v[pl.ds(j * 16, 16)]
            vals = [buf_v[b, r, pl.ds(j * 16, 16)] for r in range(CHUNK)]
            idxs = [idx16 + (256 * r) for r in range(CHUNK)]
            for r in range(CHUNK):
                plsc.addupdate_scatter(acc_v, [idxs[r]], vals[r])

        for r in range(CHUNK):
            for l in range(16):
                row = acc_v[pl.ds(r * 256 + l * 16, 16)]
                acc_v[pl.ds(r * 256 + l * 16, 16)] = zeros
                plsc.store_scatter(tmp_v, [iota * 16 + l], row)
            sums = tmp_v[pl.ds(0, 16)]
            for l in range(1, 16):
                sums = sums + tmp_v[pl.ds(l * 16, 16)]
            rates = jnp.where(
                lane_ok, jnp.where(cnt_pos, sums / safe_cnt, 0.0), -1.0
            )
            m = jnp.max(rates)
            winner = plsc.all_reduce_ffs(rates == m)
            pos = c * CHUNK + r
            plsc.store_scatter(
                out_v, [jnp.full((16,), pos, jnp.int32)], winner,
                mask=iota == 0,
            )

    @pl.loop(0, NCHUNKS, step=2)
    def _chunks(c):
        wait_dma(0)
        process(c, 0)

        @pl.when(c + 2 < NCHUNKS)
        def _():
            start_dma(c + 2, 0)

        @pl.when(c + 1 < NCHUNKS)
        def _():
            wait_dma(1)
            process(c + 1, 1)

            @pl.when(c + 3 < NCHUNKS)
            def _():
                start_dma(c + 3, 1)

    pltpu.sync_copy(out_v, out_hbm.at[pl.ds(wid * ROWS_PER_W, ROWS_PER_W)])


def _tc_body(fr_ref, asn_ref, out_ref, oh_ref, cnt_ref):
    i = pl.program_id(0)
    contract = (((1,), (1,)), ((), ()))

    @pl.when(i == 0)
    def _():
        asn_row = asn_ref[...].reshape(1, N_NEURONS)
        lbl_col = lax.broadcasted_iota(jnp.int32, (128, N_NEURONS), 0)
        oh_ref[...] = (asn_row == lbl_col).astype(jnp.bfloat16)
        cnt_ref[...] = lax.dot_general(
            jnp.ones((8, N_NEURONS), jnp.bfloat16), oh_ref[...], contract,
            preferred_element_type=jnp.float32,
        )

    ohb = oh_ref[...]

    def bdot(lhs):
        return lax.dot_general(
            lhs, ohb, contract, preferred_element_type=jnp.float32
        )

    def half_sums(x):
        c1 = x.astype(jnp.bfloat16)
        r1 = x - c1.astype(jnp.float32)
        c2 = r1.astype(jnp.bfloat16)
        r2 = r1 - c2.astype(jnp.float32)
        c3 = r2.astype(jnp.bfloat16)
        return (bdot(c3) + bdot(c2)) + bdot(c1)

    HB = BM // 2
    sums = jnp.concatenate(
        [half_sums(fr_ref[0:HB, :]), half_sums(fr_ref[HB:BM, :])], axis=0
    )
    cnt = cnt_ref[0:1, :]
    rates = jnp.where(cnt > 0.0, sums / jnp.maximum(cnt, 1.0), 0.0)
    lane = lax.broadcasted_iota(jnp.int32, (BM, 128), 1)
    rates = jnp.where(lane < N_LABELS, rates, -1.0)
    m = jnp.max(rates, axis=1, keepdims=True)
    winner = jnp.min(jnp.where(rates == m, lane, 127), axis=1)
    out_ref[...] = winner.astype(jnp.int32).reshape(1, 1, BM)


_tc_vote = pl.pallas_call(
    _tc_body,
    grid=(TC_ROWS // BM,),
    in_specs=[
        pl.BlockSpec((BM, N_NEURONS), lambda i: (i, 0)),
        pl.BlockSpec((N_NEURONS,), lambda i: (0,)),
    ],
    out_specs=pl.BlockSpec((1, 1, BM), lambda i: (i, 0, 0)),
    out_shape=jax.ShapeDtypeStruct((TC_ROWS // BM, 1, BM), jnp.int32),
    scratch_shapes=[
        pltpu.VMEM((128, N_NEURONS), jnp.bfloat16),
        pltpu.VMEM((8, 128), jnp.float32),
    ],
    compiler_params=pltpu.CompilerParams(
        dimension_semantics=("arbitrary",),
    ),
)


def kernel(firingRate, assignments):
    sc_out = _voting_kernel(firingRate, assignments)
    tc_out = _tc_vote(firingRate, assignments)
    return jnp.concatenate([tc_out.reshape(TC_ROWS), sc_out])


# speedup vs baseline: 1.0117x; 1.0117x over previous
import functools

import jax
import jax.numpy as jnp
from jax import lax
from jax.experimental import pallas as pl
from jax.experimental.pallas import tpu as pltpu
from jax.experimental.pallas import tpu_sc as plsc

N_LABELS = 10
N_NEURONS = 6400
BATCH = 4096

TC_ROWS = 2816
SC_ROWS = BATCH - TC_ROWS

NC = 2
NS = 16
NW = NC * NS
ROWS_PER_W = SC_ROWS // NW
CHUNK = 8
NCHUNKS = ROWS_PER_W // CHUNK
NVREG = N_NEURONS // 16

BM = 256
BK = 3200
NKB = N_NEURONS // BK

_mesh = plsc.VectorSubcoreMesh(
    core_axis_name="c", subcore_axis_name="s", num_cores=NC, num_subcores=NS
)


@functools.partial(
    pl.kernel,
    out_type=jax.ShapeDtypeStruct((SC_ROWS,), jnp.int32),
    mesh=_mesh,
    scratch_types=[
        pltpu.VMEM((N_NEURONS,), jnp.int32),
        pltpu.VMEM((2, CHUNK, N_NEURONS), jnp.float32),
        pltpu.VMEM((CHUNK * 256,), jnp.float32),
        pltpu.VMEM((256,), jnp.float32),
        pltpu.VMEM((ROWS_PER_W,), jnp.int32),
        pltpu.SemaphoreType.DMA,
        pltpu.SemaphoreType.DMA,
    ],
    compiler_params=pltpu.CompilerParams(needs_layout_passes=False),
)
def _voting_kernel(
    fr_hbm, asn_hbm, out_hbm, lbl_v, buf_v, acc_v, tmp_v, out_v, sem0, sem1
):
    wid = lax.axis_index("s") * NC + lax.axis_index("c")
    iota = lax.iota(jnp.int32, 16)
    zeros = jnp.zeros((16,), jnp.float32)
    ones = jnp.ones((16,), jnp.float32)
    sems = (sem0, sem1)

    def start_dma(c, b):
        row0 = TC_ROWS + wid * ROWS_PER_W + c * CHUNK
        return pltpu.async_copy(
            fr_hbm.at[pl.ds(row0, CHUNK)], buf_v.at[b], sems[b]
        )

    def wait_dma(b):
        pltpu.make_async_copy(
            fr_hbm.at[pl.ds(0, CHUNK)], buf_v.at[b], sems[b]
        ).wait()

    pltpu.sync_copy(asn_hbm, lbl_v)
    start_dma(0, 0)
    start_dma(1, 1)

    for l in range(16):
        tmp_v[pl.ds(l * 16, 16)] = zeros
    for r in range(CHUNK * 16):
        acc_v[pl.ds(r * 16, 16)] = zeros

    @pl.loop(0, NVREG)
    def _count(j):
        idx16 = lbl_v[pl.ds(j * 16, 16)] * 16 + iota
        plsc.addupdate_scatter(tmp_v, [idx16], ones)
        lbl_v[pl.ds(j * 16, 16)] = idx16

    cnt = zeros
    for l in range(16):
        row = tmp_v[pl.ds(l * 16, 16)]
        plsc.store_scatter(acc_v, [iota * 16 + l], row)
    for l in range(16):
        cnt = cnt + acc_v[pl.ds(l * 16, 16)]
    safe_cnt = jnp.maximum(cnt, 1.0)
    lane_ok = iota < N_LABELS
    cnt_pos = cnt > 0.0

    for l in range(16):
        acc_v[pl.ds(l * 16, 16)] = zeros

    def process(c, b):
        @pl.loop(0, NVREG, unroll=4)
        def _cols(j):
            idx16 = lbl_v[pl.ds(j * 16, 16)]
            vals = [buf_v[b, r, pl.ds(j * 16, 16)] for r in range(CHUNK)]
            idxs = [idx16 + (256 * r) for r in range(CHUNK)]
            for r in range(CHUNK):
                plsc.addupdate_scatter(acc_v, [idxs[r]], vals[r])

        for r in range(CHUNK):
            for l in range(16):
                row = acc_v[pl.ds(r * 256 + l * 16, 16)]
                acc_v[pl.ds(r * 256 + l * 16, 16)] = zeros
                plsc.store_scatter(tmp_v, [iota * 16 + l], row)
            sums = tmp_v[pl.ds(0, 16)]
            for l in range(1, 16):
                sums = sums + tmp_v[pl.ds(l * 16, 16)]
            rates = jnp.where(
                lane_ok, jnp.where(cnt_pos, sums / safe_cnt, 0.0), -1.0
            )
            m = jnp.max(rates)
            winner = plsc.all_reduce_ffs(rates == m)
            pos = c * CHUNK + r
            plsc.store_scatter(
                out_v, [jnp.full((16,), pos, jnp.int32)], winner,
                mask=iota == 0,
            )

    @pl.loop(0, NCHUNKS, step=2)
    def _chunks(c):
        wait_dma(0)
        process(c, 0)

        @pl.when(c + 2 < NCHUNKS)
        def _():
            start_dma(c + 2, 0)

        @pl.when(c + 1 < NCHUNKS)
        def _():
            wait_dma(1)
            process(c + 1, 1)

            @pl.when(c + 3 < NCHUNKS)
            def _():
                start_dma(c + 3, 1)

    pltpu.sync_copy(out_v, out_hbm.at[pl.ds(wid * ROWS_PER_W, ROWS_PER_W)])


def _tc_body(fr_ref, asn_ref, out_ref, oh_ref, cnt_ref):
    i = pl.program_id(0)
    contract = (((1,), (1,)), ((), ()))

    @pl.when(i == 0)
    def _():
        asn_row = asn_ref[...].reshape(1, N_NEURONS)
        lbl_col = lax.broadcasted_iota(jnp.int32, (128, N_NEURONS), 0)
        oh_ref[...] = (asn_row == lbl_col).astype(jnp.bfloat16)
        cnt_ref[...] = lax.dot_general(
            jnp.ones((8, N_NEURONS), jnp.bfloat16), oh_ref[...], contract,
            preferred_element_type=jnp.float32,
        )

    ohb = oh_ref[...]

    def bdot(lhs):
        return lax.dot_general(
            lhs, ohb, contract, preferred_element_type=jnp.float32
        )

    def half_sums(x):
        c1 = x.astype(jnp.bfloat16)
        r1 = x - c1.astype(jnp.float32)
        c2 = r1.astype(jnp.bfloat16)
        r2 = r1 - c2.astype(jnp.float32)
        c3 = r2.astype(jnp.bfloat16)
        return (bdot(c3) + bdot(c2)) + bdot(c1)

    HB = BM // 2
    sums = jnp.concatenate(
        [half_sums(fr_ref[0:HB, :]), half_sums(fr_ref[HB:BM, :])], axis=0
    )
    cnt = cnt_ref[0:1, :]
    rates = jnp.where(cnt > 0.0, sums / jnp.maximum(cnt, 1.0), 0.0)
    lane = lax.broadcasted_iota(jnp.int32, (BM, 128), 1)
    rates = jnp.where(lane < N_LABELS, rates, -1.0)
    m = jnp.max(rates, axis=1, keepdims=True)
    winner = jnp.min(jnp.where(rates == m, lane, 127), axis=1)
    out_ref[...] = winner.astype(jnp.int32).reshape(1, 1, BM)


_tc_vote = pl.pallas_call(
    _tc_body,
    grid=(TC_ROWS // BM,),
    in_specs=[
        pl.BlockSpec((BM, N_NEURONS), lambda i: (i, 0)),
        pl.BlockSpec((N_NEURONS,), lambda i: (0,)),
    ],
    out_specs=pl.BlockSpec((1, 1, BM), lambda i: (i, 0, 0)),
    out_shape=jax.ShapeDtypeStruct((TC_ROWS // BM, 1, BM), jnp.int32),
    scratch_shapes=[
        pltpu.VMEM((128, N_NEURONS), jnp.bfloat16),
        pltpu.VMEM((8, 128), jnp.float32),
    ],
    compiler_params=pltpu.CompilerParams(
        dimension_semantics=("arbitrary",),
    ),
)


def kernel(firingRate, assignments):
    sc_out = _voting_kernel(firingRate, assignments)
    tc_out = _tc_vote(firingRate, assignments)
    return jnp.concatenate([tc_out.reshape(TC_ROWS), sc_out])
